# baseline (speedup 1.0000x reference)
import functools

import jax
import jax.numpy as jnp
from jax import lax
from jax.experimental import pallas as pl
from jax.experimental.pallas import tpu as pltpu
from jax.experimental.pallas import tpu_sc as plsc

N_NODES = 100000
OUT_DIM = 128
HALF = OUT_DIM // 2
NE = 150000
PADN = 155648
ROWS_PER_ETYPE = PADN // OUT_DIM
EPT = PADN // 8
CH = 128
NCHUNK = EPT // CH
TOT = 4 * PADN
TOT_ROWS = 4 * ROWS_PER_ETYPE
REG = 0.01


def _pack_bf16_pairs(x):
    u = lax.bitcast_convert_type(x, jnp.uint32)
    b = (u + 0x7FFF + ((u >> 16) & 1)) >> 16
    lo, hi = b[..., :HALF], b[..., HALF:]
    return lax.bitcast_convert_type(lo | (hi << 16), jnp.int32)


def _tc_prep(embed):
    emb_blk = 4000
    n_blk = N_NODES // emb_blk

    def body(embed_ref, packed_ref, ssq_ref):
        i = pl.program_id(0)
        x = embed_ref[...]
        packed_ref[...] = _pack_bf16_pairs(x)

        @pl.when(i == 0)
        def _():
            ssq_ref[0, 0] = 0.0

        ssq_ref[0, 0] += jnp.sum(x * x)

    return pl.pallas_call(
        body,
        grid=(n_blk,),
        in_specs=[pl.BlockSpec((emb_blk, OUT_DIM), lambda i: (i, 0))],
        out_specs=[
            pl.BlockSpec((emb_blk, HALF), lambda i: (i, 0)),
            pl.BlockSpec(memory_space=pltpu.SMEM),
        ],
        out_shape=[
            jax.ShapeDtypeStruct((N_NODES, HALF), jnp.int32),
            jax.ShapeDtypeStruct((1, 1), jnp.float32),
        ],
    )(embed)


def _sc_scores(table, src, dst, wmat):
    mesh = plsc.VectorSubcoreMesh(core_axis_name="c", subcore_axis_name="s")

    @functools.partial(
        pl.kernel,
        mesh=mesh,
        out_type=jax.ShapeDtypeStruct((TOT,), jnp.float32),
        compiler_params=pltpu.CompilerParams(
            needs_layout_passes=False, use_tc_tiling_on_sc=False),
        scratch_types=[
            pltpu.VMEM((EPT,), jnp.int32),
            pltpu.VMEM((EPT,), jnp.int32),
            pltpu.VMEM((CH, HALF), jnp.int32),
            pltpu.VMEM((CH, HALF), jnp.int32),
            pltpu.VMEM((CH, HALF), jnp.int32),
            pltpu.VMEM((CH, HALF), jnp.int32),
            pltpu.VMEM((CH, HALF), jnp.int32),
            pltpu.VMEM((CH, HALF), jnp.int32),
            pltpu.VMEM((CH, HALF), jnp.int32),
            pltpu.VMEM((CH, HALF), jnp.int32),
            pltpu.VMEM((EPT,), jnp.float32),
            pltpu.VMEM((4, HALF), jnp.int32),
            pltpu.VMEM((CH,), jnp.int32),
            pltpu.VMEM((CH,), jnp.int32),
            pltpu.VMEM((CH,), jnp.int32),
            pltpu.VMEM((CH,), jnp.int32),
            pltpu.VMEM((CH,), jnp.int32),
            pltpu.VMEM((CH,), jnp.int32),
            pltpu.VMEM((CH,), jnp.int32),
            pltpu.VMEM((CH,), jnp.int32),
            pltpu.SemaphoreType.DMA,
            pltpu.SemaphoreType.DMA,
            pltpu.SemaphoreType.DMA,
            pltpu.SemaphoreType.DMA,
            pltpu.SemaphoreType.DMA,
            pltpu.SemaphoreType.DMA,
            pltpu.SemaphoreType.DMA,
            pltpu.SemaphoreType.DMA,
        ],
    )
    def k(table_hbm, src_hbm, dst_hbm, wmat_hbm, out_hbm,
          sidx, didx, srows0, srows1, srows2, srows3,
          orows0, orows1, orows2, orows3, scores, wrow,
          sib0, sib1, sib2, sib3, dib0, dib1, dib2, dib3,
          sem_s0, sem_o0, sem_s1, sem_o1,
          sem_s2, sem_o2, sem_s3, sem_o3):
        wid = lax.axis_index("s") * 2 + lax.axis_index("c")
        etype = wid // 8
        base = wid * EPT
        pltpu.sync_copy(wmat_hbm, wrow)
        pltpu.sync_copy(src_hbm.at[pl.ds(base, EPT)], sidx)
        pltpu.sync_copy(dst_hbm.at[pl.ds(base, EPT)], didx)
        wv = [plsc.bitcast(wrow[etype, pl.ds(kk * 16, 16)], jnp.bfloat16)
              for kk in range(4)]
        last_lane = lax.iota(jnp.int32, 16) == 15
        bufs = ((srows0, orows0, sib0, dib0, sem_s0, sem_o0),
                (srows1, orows1, sib1, dib1, sem_s1, sem_o1),
                (srows2, orows2, sib2, dib2, sem_s2, sem_o2),
                (srows3, orows3, sib3, dib3, sem_s3, sem_o3))
        nbuf = len(bufs)

        def issue(g, b):
            rs, ro, si, di, ss, so = bufs[b]
            for kk in range(CH // 16):
                si[pl.ds(kk * 16, 16)] = sidx[pl.ds(g * CH + kk * 16, 16)]
                di[pl.ds(kk * 16, 16)] = didx[pl.ds(g * CH + kk * 16, 16)]
            pltpu.async_copy(table_hbm.at[si], rs, ss)
            pltpu.async_copy(table_hbm.at[di], ro, so)

        def wait(g, b):
            rs, ro, si, di, ss, so = bufs[b]
            pltpu.make_async_copy(table_hbm.at[si], rs, ss).wait()
            pltpu.make_async_copy(table_hbm.at[di], ro, so).wait()

        for b0 in range(nbuf):
            issue(b0, b0)

        def outer(gg, carry):
            for b in range(nbuf):
                g = nbuf * gg + b
                wait(g, b)
                rs, ro = bufs[b][0], bufs[b][1]
                gbase = jnp.full((16,), g * CH, jnp.int32)

                def edge_body(e, c2):
                    fs = []
                    for kk in range(4):
                        sv = plsc.bitcast(
                            rs[e, pl.ds(kk * 16, 16)], jnp.bfloat16)
                        ov = plsc.bitcast(
                            ro[e, pl.ds(kk * 16, 16)], jnp.bfloat16)
                        p = (sv * wv[kk]) * ov
                        lo, hi = plsc.unpack(
                            p, format=plsc.PackFormat.INTERLEAVED)
                        fs.append(lo + hi)
                    tot = jnp.full(
                        (16,), jnp.sum((fs[0] + fs[1]) + (fs[2] + fs[3])))
                    plsc.store_scatter(
                        scores, [gbase + e], tot, mask=last_lane)
                    return c2

                lax.fori_loop(0, CH, edge_body, 0)

                @pl.when(g + nbuf < NCHUNK)
                def _():
                    issue(g + nbuf, b)
            return carry

        lax.fori_loop(0, NCHUNK // nbuf, outer, 0)
        pltpu.sync_copy(scores, out_hbm.at[pl.ds(base, EPT)])

    return k(table, src, dst, wmat)


def _tc_loss(scores4, labels4, wmat, ssq):
    def body(scores_ref, labels_ref, wmat_ref, ssq_ref, out_ref):
        x = scores_ref[...]
        y = labels_ref[...]
        row = lax.broadcasted_iota(jnp.int32, x.shape, 0)
        col = lax.broadcasted_iota(jnp.int32, x.shape, 1)
        rin = row % ROWS_PER_ETYPE
        valid = (rin * OUT_DIM + col) < NE
        bce = jnp.maximum(x, 0.0) - x * y + jnp.log1p(jnp.exp(-jnp.abs(x)))
        bce = jnp.where(valid, bce, 0.0)
        w = wmat_ref[...]
        reg = ssq_ref[0, 0] / (N_NODES * OUT_DIM) + jnp.sum(w * w) / OUT_DIM
        out_ref[0, 0] = jnp.sum(bce) / NE + REG * reg

    out = pl.pallas_call(
        body,
        in_specs=[
            pl.BlockSpec((TOT_ROWS, OUT_DIM), lambda: (0, 0)),
            pl.BlockSpec((TOT_ROWS, OUT_DIM), lambda: (0, 0)),
            pl.BlockSpec((4, OUT_DIM), lambda: (0, 0)),
            pl.BlockSpec(memory_space=pltpu.SMEM),
        ],
        out_specs=pl.BlockSpec(memory_space=pltpu.SMEM),
        out_shape=jax.ShapeDtypeStruct((1, 1), jnp.float32),
    )(scores4, labels4, wmat, ssq)
    return out[0, 0]


def kernel(embed_0,
           edges_rel0, edges_rel1, edges_rel2, edges_rel3,
           labels_rel0, labels_rel1, labels_rel2, labels_rel3,
           w_rel0, w_rel1, w_rel2, w_rel3):
    edges = [edges_rel0, edges_rel1, edges_rel2, edges_rel3]
    labels = [labels_rel0, labels_rel1, labels_rel2, labels_rel3]
    pad = PADN - NE
    pad_idx = (jnp.arange(pad, dtype=jnp.int32) * 17) % N_NODES
    src = jnp.concatenate(
        [jnp.concatenate([ed[:, 0], pad_idx]) for ed in edges])
    dst = jnp.concatenate(
        [jnp.concatenate([ed[:, 1], pad_idx]) for ed in edges])
    lab = jnp.concatenate([jnp.pad(lb, (0, pad)) for lb in labels])
    wmat = jnp.stack([w_rel0, w_rel1, w_rel2, w_rel3])

    packed, ssq = _tc_prep(embed_0)
    scores = _sc_scores(packed, src, dst, _pack_bf16_pairs(wmat))
    return _tc_loss(scores.reshape(TOT_ROWS, OUT_DIM),
                    lab.reshape(TOT_ROWS, OUT_DIM),
                    wmat, ssq)


# speedup vs baseline: 1.5921x; 1.5921x over previous
import functools

import jax
import jax.numpy as jnp
from jax import lax
from jax.experimental import pallas as pl
from jax.experimental.pallas import tpu as pltpu
from jax.experimental.pallas import tpu_sc as plsc

N_NODES = 100000
OUT_DIM = 128
HALF = OUT_DIM // 2
NE = 150000
PADN = 155648
ROWS_PER_ETYPE = PADN // OUT_DIM
EPT = PADN // 8
CH = 128
NCHUNK = EPT // CH
TOT = 4 * PADN
TOT_ROWS = 4 * ROWS_PER_ETYPE
REG = 0.01


def _pack_bf16_pairs(x):
    u = lax.bitcast_convert_type(x, jnp.uint32)
    b = (u + 0x7FFF + ((u >> 16) & 1)) >> 16
    lo, hi = b[..., :HALF], b[..., HALF:]
    return lax.bitcast_convert_type(lo | (hi << 16), jnp.int32)


def _tc_prep(embed):
    emb_blk = 4000
    n_blk = N_NODES // emb_blk

    def body(embed_ref, packed_ref, ssq_ref):
        i = pl.program_id(0)
        x = embed_ref[...]
        packed_ref[...] = _pack_bf16_pairs(x)

        @pl.when(i == 0)
        def _():
            ssq_ref[0, 0] = 0.0

        ssq_ref[0, 0] += jnp.sum(x * x)

    return pl.pallas_call(
        body,
        grid=(n_blk,),
        in_specs=[pl.BlockSpec((emb_blk, OUT_DIM), lambda i: (i, 0))],
        out_specs=[
            pl.BlockSpec((emb_blk, HALF), lambda i: (i, 0)),
            pl.BlockSpec(memory_space=pltpu.SMEM),
        ],
        out_shape=[
            jax.ShapeDtypeStruct((N_NODES, HALF), jnp.int32),
            jax.ShapeDtypeStruct((1, 1), jnp.float32),
        ],
    )(embed)


def _sc_scores(table, src, dst, wmat):
    mesh = plsc.VectorSubcoreMesh(core_axis_name="c", subcore_axis_name="s")

    @functools.partial(
        pl.kernel,
        mesh=mesh,
        out_type=jax.ShapeDtypeStruct((TOT,), jnp.float32),
        compiler_params=pltpu.CompilerParams(
            needs_layout_passes=False, use_tc_tiling_on_sc=False),
        scratch_types=[
            pltpu.VMEM((EPT,), jnp.int32),
            pltpu.VMEM((EPT,), jnp.int32),
            pltpu.VMEM((CH, HALF), jnp.int32),
            pltpu.VMEM((CH, HALF), jnp.int32),
            pltpu.VMEM((CH, HALF), jnp.int32),
            pltpu.VMEM((CH, HALF), jnp.int32),
            pltpu.VMEM((CH, HALF), jnp.int32),
            pltpu.VMEM((CH, HALF), jnp.int32),
            pltpu.VMEM((CH, HALF), jnp.int32),
            pltpu.VMEM((CH, HALF), jnp.int32),
            pltpu.VMEM((EPT,), jnp.float32),
            pltpu.VMEM((4, HALF), jnp.int32),
            pltpu.VMEM((CH,), jnp.int32),
            pltpu.VMEM((CH,), jnp.int32),
            pltpu.VMEM((CH,), jnp.int32),
            pltpu.VMEM((CH,), jnp.int32),
            pltpu.VMEM((CH,), jnp.int32),
            pltpu.VMEM((CH,), jnp.int32),
            pltpu.VMEM((CH,), jnp.int32),
            pltpu.VMEM((CH,), jnp.int32),
            pltpu.SemaphoreType.DMA,
            pltpu.SemaphoreType.DMA,
            pltpu.SemaphoreType.DMA,
            pltpu.SemaphoreType.DMA,
            pltpu.SemaphoreType.DMA,
            pltpu.SemaphoreType.DMA,
            pltpu.SemaphoreType.DMA,
            pltpu.SemaphoreType.DMA,
        ],
    )
    def k(table_hbm, src_hbm, dst_hbm, wmat_hbm, out_hbm,
          sidx, didx, srows0, srows1, srows2, srows3,
          orows0, orows1, orows2, orows3, scores, wrow,
          sib0, sib1, sib2, sib3, dib0, dib1, dib2, dib3,
          sem_s0, sem_o0, sem_s1, sem_o1,
          sem_s2, sem_o2, sem_s3, sem_o3):
        wid = lax.axis_index("s") * 2 + lax.axis_index("c")
        etype = wid // 8
        base = wid * EPT
        pltpu.sync_copy(wmat_hbm, wrow)
        pltpu.sync_copy(src_hbm.at[pl.ds(base, EPT)], sidx)
        pltpu.sync_copy(dst_hbm.at[pl.ds(base, EPT)], didx)
        wv = [plsc.bitcast(wrow[etype, pl.ds(kk * 16, 16)], jnp.bfloat16)
              for kk in range(4)]
        lane = lax.iota(jnp.int32, 16)
        xor_idx = {k: lane ^ k for k in (8, 4, 2, 1)}
        grp_mask = {k: (lane & k) == 0 for k in (8, 4, 2, 1)}

        def fold(v, k):
            return v + v.at[xor_idx[k]].get(mode="promise_in_bounds")
        bufs = ((srows0, orows0, sib0, dib0, sem_s0, sem_o0),
                (srows1, orows1, sib1, dib1, sem_s1, sem_o1),
                (srows2, orows2, sib2, dib2, sem_s2, sem_o2),
                (srows3, orows3, sib3, dib3, sem_s3, sem_o3))
        nbuf = len(bufs)

        def issue(g, b):
            rs, ro, si, di, ss, so = bufs[b]
            for kk in range(CH // 16):
                si[pl.ds(kk * 16, 16)] = sidx[pl.ds(g * CH + kk * 16, 16)]
                di[pl.ds(kk * 16, 16)] = didx[pl.ds(g * CH + kk * 16, 16)]
            pltpu.async_copy(table_hbm.at[si], rs, ss)
            pltpu.async_copy(table_hbm.at[di], ro, so)

        def wait(g, b):
            rs, ro, si, di, ss, so = bufs[b]
            pltpu.make_async_copy(table_hbm.at[si], rs, ss).wait()
            pltpu.make_async_copy(table_hbm.at[di], ro, so).wait()

        for b0 in range(nbuf):
            issue(b0, b0)

        def outer(gg, carry):
            for b in range(nbuf):
                g = nbuf * gg + b
                wait(g, b)
                rs, ro = bufs[b][0], bufs[b][1]

                def group_body(j, c2):
                    ebase = j * 16
                    accs = []
                    for t in range(16):
                        e = ebase + t
                        fs = []
                        for kk in range(4):
                            sv = plsc.bitcast(
                                rs[e, pl.ds(kk * 16, 16)], jnp.bfloat16)
                            ov = plsc.bitcast(
                                ro[e, pl.ds(kk * 16, 16)], jnp.bfloat16)
                            p = (sv * wv[kk]) * ov
                            lo, hi = plsc.unpack(
                                p, format=plsc.PackFormat.INTERLEAVED)
                            fs.append(lo + hi)
                        accs.append((fs[0] + fs[1]) + (fs[2] + fs[3]))
                    m1 = [jnp.where(grp_mask[8], fold(accs[a], 8),
                                    fold(accs[a + 8], 8)) for a in range(8)]
                    m2 = [jnp.where(grp_mask[4], fold(m1[a], 4),
                                    fold(m1[a + 4], 4)) for a in range(4)]
                    m3 = [jnp.where(grp_mask[2], fold(m2[a], 2),
                                    fold(m2[a + 2], 2)) for a in range(2)]
                    s16 = jnp.where(grp_mask[1], fold(m3[0], 1),
                                    fold(m3[1], 1))
                    scores[pl.ds(g * CH + ebase, 16)] = s16
                    return c2

                lax.fori_loop(0, CH // 16, group_body, 0)

                @pl.when(g + nbuf < NCHUNK)
                def _():
                    issue(g + nbuf, b)
            return carry

        lax.fori_loop(0, NCHUNK // nbuf, outer, 0)
        pltpu.sync_copy(scores, out_hbm.at[pl.ds(base, EPT)])

    return k(table, src, dst, wmat)


def _tc_loss(scores4, labels4, wmat, ssq):
    def body(scores_ref, labels_ref, wmat_ref, ssq_ref, out_ref):
        x = scores_ref[...]
        y = labels_ref[...]
        row = lax.broadcasted_iota(jnp.int32, x.shape, 0)
        col = lax.broadcasted_iota(jnp.int32, x.shape, 1)
        rin = row % ROWS_PER_ETYPE
        valid = (rin * OUT_DIM + col) < NE
        bce = jnp.maximum(x, 0.0) - x * y + jnp.log1p(jnp.exp(-jnp.abs(x)))
        bce = jnp.where(valid, bce, 0.0)
        w = wmat_ref[...]
        reg = ssq_ref[0, 0] / (N_NODES * OUT_DIM) + jnp.sum(w * w) / OUT_DIM
        out_ref[0, 0] = jnp.sum(bce) / NE + REG * reg

    out = pl.pallas_call(
        body,
        in_specs=[
            pl.BlockSpec((TOT_ROWS, OUT_DIM), lambda: (0, 0)),
            pl.BlockSpec((TOT_ROWS, OUT_DIM), lambda: (0, 0)),
            pl.BlockSpec((4, OUT_DIM), lambda: (0, 0)),
            pl.BlockSpec(memory_space=pltpu.SMEM),
        ],
        out_specs=pl.BlockSpec(memory_space=pltpu.SMEM),
        out_shape=jax.ShapeDtypeStruct((1, 1), jnp.float32),
    )(scores4, labels4, wmat, ssq)
    return out[0, 0]


def kernel(embed_0,
           edges_rel0, edges_rel1, edges_rel2, edges_rel3,
           labels_rel0, labels_rel1, labels_rel2, labels_rel3,
           w_rel0, w_rel1, w_rel2, w_rel3):
    edges = [edges_rel0, edges_rel1, edges_rel2, edges_rel3]
    labels = [labels_rel0, labels_rel1, labels_rel2, labels_rel3]
    pad = PADN - NE
    pad_idx = (jnp.arange(pad, dtype=jnp.int32) * 17) % N_NODES
    src = jnp.concatenate(
        [jnp.concatenate([ed[:, 0], pad_idx]) for ed in edges])
    dst = jnp.concatenate(
        [jnp.concatenate([ed[:, 1], pad_idx]) for ed in edges])
    lab = jnp.concatenate([jnp.pad(lb, (0, pad)) for lb in labels])
    wmat = jnp.stack([w_rel0, w_rel1, w_rel2, w_rel3])

    packed, ssq = _tc_prep(embed_0)
    scores = _sc_scores(packed, src, dst, _pack_bf16_pairs(wmat))
    return _tc_loss(scores.reshape(TOT_ROWS, OUT_DIM),
                    lab.reshape(TOT_ROWS, OUT_DIM),
                    wmat, ssq)


# speedup vs baseline: 1.7031x; 1.0697x over previous
import functools

import jax
import jax.numpy as jnp
from jax import lax
from jax.experimental import pallas as pl
from jax.experimental.pallas import tpu as pltpu
from jax.experimental.pallas import tpu_sc as plsc

N_NODES = 100000
OUT_DIM = 128
HALF = OUT_DIM // 2
NE = 150000
PADN = 155648
ROWS_PER_ETYPE = PADN // OUT_DIM
EPT = PADN // 8
CH = 128
NCHUNK = EPT // CH
TOT = 4 * PADN
TOT_ROWS = 4 * ROWS_PER_ETYPE
REG = 0.01


def _pack_bf16_pairs(x):
    u = lax.bitcast_convert_type(x, jnp.uint32)
    b = (u + 0x7FFF + ((u >> 16) & 1)) >> 16
    lo, hi = b[..., :HALF], b[..., HALF:]
    return lax.bitcast_convert_type(lo | (hi << 16), jnp.int32)


def _tc_prep(embed):
    emb_blk = 2000
    n_blk = N_NODES // 2 // emb_blk

    def body(lo_ref, hi_ref, packed_ref, ssq_ref):
        i = pl.program_id(0)
        xa = lo_ref[...]
        xb = hi_ref[...]
        packed_ref[...] = jnp.concatenate(
            [_pack_bf16_pairs(xa), _pack_bf16_pairs(xb)], axis=1)

        @pl.when(i == 0)
        def _():
            ssq_ref[0, 0] = 0.0

        ssq_ref[0, 0] += jnp.sum(xa * xa) + jnp.sum(xb * xb)

    return pl.pallas_call(
        body,
        grid=(n_blk,),
        in_specs=[
            pl.BlockSpec((emb_blk, OUT_DIM), lambda i: (i, 0)),
            pl.BlockSpec((emb_blk, OUT_DIM), lambda i: (i + 25, 0)),
        ],
        out_specs=[
            pl.BlockSpec((emb_blk, OUT_DIM), lambda i: (i, 0)),
            pl.BlockSpec(memory_space=pltpu.SMEM),
        ],
        out_shape=[
            jax.ShapeDtypeStruct((N_NODES // 2, OUT_DIM), jnp.int32),
            jax.ShapeDtypeStruct((1, 1), jnp.float32),
        ],
    )(embed, embed)


def _sc_scores(table, src, dst, wmat):
    mesh = plsc.VectorSubcoreMesh(core_axis_name="c", subcore_axis_name="s")

    @functools.partial(
        pl.kernel,
        mesh=mesh,
        out_type=jax.ShapeDtypeStruct((TOT,), jnp.float32),
        compiler_params=pltpu.CompilerParams(
            needs_layout_passes=False, use_tc_tiling_on_sc=False),
        scratch_types=[
            pltpu.VMEM((EPT,), jnp.int32),
            pltpu.VMEM((EPT,), jnp.int32),
            pltpu.VMEM((CH, HALF), jnp.int32),
            pltpu.VMEM((CH, HALF), jnp.int32),
            pltpu.VMEM((CH, HALF), jnp.int32),
            pltpu.VMEM((CH, HALF), jnp.int32),
            pltpu.VMEM((CH, HALF), jnp.int32),
            pltpu.VMEM((CH, HALF), jnp.int32),
            pltpu.VMEM((CH, HALF), jnp.int32),
            pltpu.VMEM((CH, HALF), jnp.int32),
            pltpu.VMEM((EPT,), jnp.float32),
            pltpu.VMEM((4, HALF), jnp.int32),
            pltpu.VMEM((CH,), jnp.int32),
            pltpu.VMEM((CH,), jnp.int32),
            pltpu.VMEM((CH,), jnp.int32),
            pltpu.VMEM((CH,), jnp.int32),
            pltpu.VMEM((CH,), jnp.int32),
            pltpu.VMEM((CH,), jnp.int32),
            pltpu.VMEM((CH,), jnp.int32),
            pltpu.VMEM((CH,), jnp.int32),
            pltpu.SemaphoreType.DMA,
            pltpu.SemaphoreType.DMA,
            pltpu.SemaphoreType.DMA,
            pltpu.SemaphoreType.DMA,
            pltpu.SemaphoreType.DMA,
            pltpu.SemaphoreType.DMA,
            pltpu.SemaphoreType.DMA,
            pltpu.SemaphoreType.DMA,
        ],
    )
    def k(table_hbm, src_hbm, dst_hbm, wmat_hbm, out_hbm,
          sidx, didx, srows0, srows1, srows2, srows3,
          orows0, orows1, orows2, orows3, scores, wrow,
          sib0, sib1, sib2, sib3, dib0, dib1, dib2, dib3,
          sem_s0, sem_o0, sem_s1, sem_o1,
          sem_s2, sem_o2, sem_s3, sem_o3):
        wid = lax.axis_index("s") * 2 + lax.axis_index("c")
        etype = wid // 8
        base = wid * EPT
        pltpu.sync_copy(wmat_hbm, wrow)
        pltpu.sync_copy(src_hbm.at[pl.ds(base, EPT)], sidx)
        pltpu.sync_copy(dst_hbm.at[pl.ds(base, EPT)], didx)
        wv = [plsc.bitcast(wrow[etype, pl.ds(kk * 16, 16)], jnp.bfloat16)
              for kk in range(4)]
        lane = lax.iota(jnp.int32, 16)
        xor_idx = {k: lane ^ k for k in (8, 4, 2, 1)}
        grp_mask = {k: (lane & k) == 0 for k in (8, 4, 2, 1)}

        def fold(v, k):
            return v + v.at[xor_idx[k]].get(mode="promise_in_bounds")
        bufs = ((srows0, orows0, sib0, dib0, sem_s0, sem_o0),
                (srows1, orows1, sib1, dib1, sem_s1, sem_o1),
                (srows2, orows2, sib2, dib2, sem_s2, sem_o2),
                (srows3, orows3, sib3, dib3, sem_s3, sem_o3))
        nbuf = len(bufs)

        def remap(v):
            return jnp.where(v >= N_NODES // 2,
                             2 * v - (N_NODES - 1), 2 * v)

        def issue(g, b):
            rs, ro, si, di, ss, so = bufs[b]
            for kk in range(CH // 16):
                si[pl.ds(kk * 16, 16)] = remap(
                    sidx[pl.ds(g * CH + kk * 16, 16)])
                di[pl.ds(kk * 16, 16)] = remap(
                    didx[pl.ds(g * CH + kk * 16, 16)])
            pltpu.async_copy(table_hbm.at[si], rs, ss)
            pltpu.async_copy(table_hbm.at[di], ro, so)

        def wait(g, b):
            rs, ro, si, di, ss, so = bufs[b]
            pltpu.make_async_copy(table_hbm.at[si], rs, ss).wait()
            pltpu.make_async_copy(table_hbm.at[di], ro, so).wait()

        for b0 in range(nbuf):
            issue(b0, b0)

        def outer(gg, carry):
            for b in range(nbuf):
                g = nbuf * gg + b
                wait(g, b)
                rs, ro = bufs[b][0], bufs[b][1]

                def group_body(j, c2):
                    ebase = j * 16
                    accs = []
                    for t in range(16):
                        e = ebase + t
                        fs = []
                        for kk in range(4):
                            sv = plsc.bitcast(
                                rs[e, pl.ds(kk * 16, 16)], jnp.bfloat16)
                            ov = plsc.bitcast(
                                ro[e, pl.ds(kk * 16, 16)], jnp.bfloat16)
                            p = (sv * wv[kk]) * ov
                            lo, hi = plsc.unpack(
                                p, format=plsc.PackFormat.INTERLEAVED)
                            fs.append(lo + hi)
                        accs.append((fs[0] + fs[1]) + (fs[2] + fs[3]))
                    m1 = [jnp.where(grp_mask[8], fold(accs[a], 8),
                                    fold(accs[a + 8], 8)) for a in range(8)]
                    m2 = [jnp.where(grp_mask[4], fold(m1[a], 4),
                                    fold(m1[a + 4], 4)) for a in range(4)]
                    m3 = [jnp.where(grp_mask[2], fold(m2[a], 2),
                                    fold(m2[a + 2], 2)) for a in range(2)]
                    s16 = jnp.where(grp_mask[1], fold(m3[0], 1),
                                    fold(m3[1], 1))
                    scores[pl.ds(g * CH + ebase, 16)] = s16
                    return c2

                lax.fori_loop(0, CH // 16, group_body, 0)

                @pl.when(g + nbuf < NCHUNK)
                def _():
                    issue(g + nbuf, b)
            return carry

        lax.fori_loop(0, NCHUNK // nbuf, outer, 0)
        pltpu.sync_copy(scores, out_hbm.at[pl.ds(base, EPT)])

    return k(table, src, dst, wmat)


def _tc_loss(scores4, labels4, wmat, ssq):
    def body(scores_ref, labels_ref, wmat_ref, ssq_ref, out_ref):
        x = scores_ref[...]
        y = labels_ref[...]
        row = lax.broadcasted_iota(jnp.int32, x.shape, 0)
        col = lax.broadcasted_iota(jnp.int32, x.shape, 1)
        rin = row % ROWS_PER_ETYPE
        valid = (rin * OUT_DIM + col) < NE
        bce = jnp.maximum(x, 0.0) - x * y + jnp.log1p(jnp.exp(-jnp.abs(x)))
        bce = jnp.where(valid, bce, 0.0)
        w = wmat_ref[...]
        reg = ssq_ref[0, 0] / (N_NODES * OUT_DIM) + jnp.sum(w * w) / OUT_DIM
        out_ref[0, 0] = jnp.sum(bce) / NE + REG * reg

    out = pl.pallas_call(
        body,
        in_specs=[
            pl.BlockSpec((TOT_ROWS, OUT_DIM), lambda: (0, 0)),
            pl.BlockSpec((TOT_ROWS, OUT_DIM), lambda: (0, 0)),
            pl.BlockSpec((4, OUT_DIM), lambda: (0, 0)),
            pl.BlockSpec(memory_space=pltpu.SMEM),
        ],
        out_specs=pl.BlockSpec(memory_space=pltpu.SMEM),
        out_shape=jax.ShapeDtypeStruct((1, 1), jnp.float32),
    )(scores4, labels4, wmat, ssq)
    return out[0, 0]


def kernel(embed_0,
           edges_rel0, edges_rel1, edges_rel2, edges_rel3,
           labels_rel0, labels_rel1, labels_rel2, labels_rel3,
           w_rel0, w_rel1, w_rel2, w_rel3):
    edges = [edges_rel0, edges_rel1, edges_rel2, edges_rel3]
    labels = [labels_rel0, labels_rel1, labels_rel2, labels_rel3]
    pad = PADN - NE
    pad_idx = (jnp.arange(pad, dtype=jnp.int32) * 17) % N_NODES
    src = jnp.concatenate(
        [jnp.concatenate([ed[:, 0], pad_idx]) for ed in edges])
    dst = jnp.concatenate(
        [jnp.concatenate([ed[:, 1], pad_idx]) for ed in edges])
    lab = jnp.concatenate([jnp.pad(lb, (0, pad)) for lb in labels])
    wmat = jnp.stack([w_rel0, w_rel1, w_rel2, w_rel3])

    packed, ssq = _tc_prep(embed_0)
    scores = _sc_scores(packed.reshape(N_NODES, HALF), src, dst,
                        _pack_bf16_pairs(wmat))
    return _tc_loss(scores.reshape(TOT_ROWS, OUT_DIM),
                    lab.reshape(TOT_ROWS, OUT_DIM),
                    wmat, ssq)
